# baseline (speedup 1.0000x reference)
import functools

import jax
import jax.numpy as jnp
from jax import lax
from jax.experimental import pallas as pl
from jax.experimental.pallas import tpu as pltpu
from jax.experimental.pallas import tpu_sc as plsc

SLATE = 50
B, H, V = 64, 8, 100000


VB = 8192
NB = 13
VP = NB * VB
NEG = -3.0e38


def _scores_body(x_ref, o_ref):
    x = x_ref[...]
    l = jnp.log(x + jnp.float32(1e-7))
    acc = l[:, 0]
    for h in range(1, H):
        acc = acc + l[:, h]
    col = pl.program_id(1) * VB + lax.broadcasted_iota(jnp.int32, (1, VB), 1)
    acc = jnp.where(col < V, acc, jnp.float32(NEG))
    o_ref[...] = acc.reshape(VB // 128, 128)


def _scores(x, half, hb):
    return pl.pallas_call(
        _scores_body,
        grid=(hb, NB),
        in_specs=[pl.BlockSpec(
            (1, H, VB), lambda b, j, half=half: (b + half * hb, 0, j))],
        out_specs=pl.BlockSpec(
            (VB // 128, 128), lambda b, j: (b * NB + j, 0)),
        out_shape=jax.ShapeDtypeStruct((hb * (VP // 128), 128), jnp.float32),
    )(x)


L = 16
NC, NS = 2, 16
ROWS_PER_W = B // (NC * NS)
CHUNKS = [(0, 1024, 8), (1024, 3072, 8), (4096, 20480, 8),
          (24576, 20480, 8), (45056, 20480, 8), (65536, 20480, 8),
          (86016, 20480, 8)]
DROWS = 160
CAP = 20480 + 64
IMIN = -2147483648
IMAX = 2147483647
KPAD = 64


def _splat(x, dtype=jnp.int32):
    return jnp.full((L,), x, dtype)


def _mono_key(v):
    bits = lax.bitcast_convert_type(v, jnp.int32)
    return jnp.where(bits < 0, IMIN - bits, bits)


def _count_pass(src, n, pivot, iota):
    nv = (n + L - 1) // L
    ps = jnp.broadcast_to(pivot, (L,))
    ns = jnp.broadcast_to(n, (L,))

    def body(i, carry):
        cg, ce = carry
        k = src[pl.ds(i * L, L)]
        valid = (iota + i * L) < ns
        g = (k > ps) & valid
        e = (k == ps) & valid
        return (cg + plsc.all_reduce_population_count(g),
                ce + plsc.all_reduce_population_count(e))

    cg, ce = lax.fori_loop(0, nv, body, (_splat(0), _splat(0)))
    return jnp.max(cg), jnp.max(ce)


def _filter_pass(src, dst, n, pivot, keep_gt, iota):
    nv = (n + L - 1) // L
    ps = jnp.broadcast_to(pivot, (L,))
    ns = jnp.broadcast_to(n, (L,))
    kgb = jnp.broadcast_to(keep_gt, (L,))

    def body(i, pos):
        k = src[pl.ds(i * L, L)]
        valid = (iota + i * L) < ns
        m = jnp.where(kgb, k > ps, k < ps) & valid
        pc = plsc.cumsum(m.astype(jnp.int32))
        tgt = pos + pc - 1
        plsc.store_scatter(dst, [tgt], k, mask=m)
        return pos + plsc.all_reduce_population_count(m)

    pos = lax.fori_loop(0, nv, body, _splat(0))
    return jnp.max(pos)


def _lane0(vec):
    iota = lax.iota(jnp.int32, L)
    return jnp.max(jnp.where(iota == 0, vec, _splat(IMIN)))


def _rebuild(cand_k, cand_i, surv, kept_k, kept_i, m, iota):
    need0 = jnp.int32(SLATE)

    def qs_iter(src, dst, n, need):
        pivot = _lane0(src[pl.ds(0, L)])
        cg, ce = _count_pass(src, n, pivot, iota)
        done = (cg < need) & (cg + ce >= need)
        keep_gt = cg >= need
        new_need = jnp.where(keep_gt, need, need - cg - ce)
        new_n = lax.cond(
            done, lambda _: n,
            lambda _: _filter_pass(src, dst, n, pivot, keep_gt, iota), 0)
        return new_n, new_need, pivot, done

    n1, need1, t1, done1 = qs_iter(cand_k, surv, m, need0)

    def w_cond(c):
        return jnp.logical_not(c[3])

    def w_body(c):
        n, need, _, _ = c
        return qs_iter(surv, surv, n, need)

    _, _, T, _ = lax.while_loop(w_cond, w_body, (n1, need1, t1, done1))

    r, _ = _count_pass(cand_k, m, T, iota)
    quota = jnp.broadcast_to(SLATE - r, (L,))
    Ts = jnp.broadcast_to(T, (L,))
    ms = jnp.broadcast_to(m, (L,))
    nv = (m + L - 1) // L

    def comp_body(i, carry):
        pos, eqc = carry
        k = cand_k[pl.ds(i * L, L)]
        ix = cand_i[pl.ds(i * L, L)]
        valid = (iota + i * L) < ms
        g = (k > Ts) & valid
        e = (k == Ts) & valid
        eseq = eqc + plsc.cumsum(e.astype(jnp.int32))
        keep = g | (e & (eseq <= quota))
        pc = plsc.cumsum(keep.astype(jnp.int32))
        tgt = pos + pc - 1
        plsc.store_scatter(kept_k, [tgt], k, mask=keep)
        plsc.store_scatter(kept_i, [tgt], ix, mask=keep)
        return (pos + plsc.all_reduce_population_count(keep),
                eqc + plsc.all_reduce_population_count(e))

    lax.fori_loop(0, nv, comp_body, (_splat(0), _splat(0)))
    plsc.store_scatter(kept_k, [iota + 48], _splat(IMIN), mask=iota >= 2)
    return T


def _topk_sc_body(rows_per_w, scores_hbm, out_hbm, data_a, data_b,
                  cand_k, cand_i, surv, kept_k, kept_i, outb, sem_a, sem_b):
    wid = lax.axis_index("s") * NC + lax.axis_index("c")
    iota = lax.iota(jnp.int32, L)
    bufs = (data_a, data_b)
    sems = (sem_a, sem_b)

    def row_body(rr, carry):
        row = wid * rows_per_w + rr
        for j in range(KPAD // L):
            kept_k[pl.ds(L * j, L)] = _splat(IMIN)
            kept_i[pl.ds(L * j, L)] = _splat(0)
        T = IMIN
        off0, size0, _ = CHUNKS[0]
        h = pltpu.async_copy(
            scores_hbm.at[pl.ds(row * (VP // 128) + off0 // 128,
                                size0 // 128), :],
            bufs[0].at[pl.ds(0, size0 // 128), :], sems[0])
        for ci, (off, size, unroll) in enumerate(CHUNKS):
            data = bufs[ci % 2]
            if ci + 1 < len(CHUNKS):
                noff, nsize, _ = CHUNKS[ci + 1]
                hn = pltpu.async_copy(
                    scores_hbm.at[pl.ds(row * (VP // 128) + noff // 128,
                                        nsize // 128), :],
                    bufs[(ci + 1) % 2].at[pl.ds(0, nsize // 128), :],
                    sems[(ci + 1) % 2])
            h.wait()
            for j in range(KPAD // L):
                cand_k[pl.ds(L * j, L)] = kept_k[pl.ds(L * j, L)]
                cand_i[pl.ds(L * j, L)] = kept_i[pl.ds(L * j, L)]
            Ts = jnp.broadcast_to(T, (L,))

            def app_body(i, pos, off=off, data=data, Ts=Ts, unroll=unroll):
                base = i * (L * unroll)
                for u in range(unroll):
                    f = i * (unroll // 8) + u // 8
                    v = data[f, pl.ds((u % 8) * L, L)]
                    key = _mono_key(v)
                    anyh = plsc.all_reduce_population_count(key > Ts) > 0
                    tgt = pos + iota
                    plsc.store_scatter(cand_k, [tgt], key)
                    gidx = iota + (off + u * L) + base
                    plsc.store_scatter(cand_i, [tgt], gidx)
                    pos = pos + jnp.where(anyh, L, 0)
                return pos

            pos = lax.fori_loop(0, size // (L * unroll), app_body,
                                _splat(KPAD))
            m = jnp.max(pos)
            T = _rebuild(cand_k, cand_i, surv, kept_k, kept_i, m, iota)
            if ci + 1 < len(CHUNKS):
                h = hn

        ks = [kept_k[pl.ds(L * j, L)] for j in range(KPAD // L)]
        ix = [kept_i[pl.ds(L * j, L)] for j in range(KPAD // L)]
        for t in range(SLATE):
            m4 = jnp.maximum(jnp.maximum(ks[0], ks[1]),
                             jnp.maximum(ks[2], ks[3]))
            Mv = jnp.max(m4)
            Ms = jnp.broadcast_to(Mv, (L,))
            cands = [jnp.where(ks[j] == Ms, ix[j], _splat(IMAX))
                     for j in range(KPAD // L)]
            mn = jnp.min(jnp.minimum(jnp.minimum(cands[0], cands[1]),
                                     jnp.minimum(cands[2], cands[3])))
            mns = jnp.broadcast_to(mn, (L,))
            plsc.store_scatter(outb, [_splat(t)], mns, mask=iota == 0)
            ks = [jnp.where((ks[j] == Ms) & (ix[j] == mns), _splat(IMIN),
                            ks[j]) for j in range(KPAD // L)]
        plsc.store_scatter(outb, [iota + 48], _splat(0), mask=iota >= 2)
        pltpu.sync_copy(outb, out_hbm.at[pl.ds(row * KPAD, KPAD)])
        return carry

    lax.fori_loop(0, rows_per_w, row_body, jnp.int32(0))


def _topk_sc(scores, nrows):
    mesh = plsc.VectorSubcoreMesh(core_axis_name="c", subcore_axis_name="s")
    f = functools.partial(
        pl.kernel, mesh=mesh,
        compiler_params=pltpu.CompilerParams(needs_layout_passes=False),
        out_type=jax.ShapeDtypeStruct((nrows * KPAD,), jnp.int32),
        scratch_types=[
            pltpu.VMEM((DROWS, 128), jnp.float32),
            pltpu.VMEM((DROWS, 128), jnp.float32),
            pltpu.VMEM((CAP,), jnp.int32),
            pltpu.VMEM((CAP,), jnp.int32),
            pltpu.VMEM((CAP,), jnp.int32),
            pltpu.VMEM((KPAD,), jnp.int32),
            pltpu.VMEM((KPAD,), jnp.int32),
            pltpu.VMEM((KPAD,), jnp.int32),
            pltpu.SemaphoreType.DMA,
            pltpu.SemaphoreType.DMA,
        ],
    )(functools.partial(_topk_sc_body, nrows // (NC * NS)))
    return f(scores)


def kernel(batch_k_head_softmax):
    hb = B // 2
    s0 = _scores(batch_k_head_softmax, 0, hb)
    o0 = _topk_sc(s0, hb)
    s1 = _scores(batch_k_head_softmax, 1, hb)
    o1 = _topk_sc(s1, hb)
    out = jnp.concatenate([o0.reshape(hb, KPAD), o1.reshape(hb, KPAD)], 0)
    return out[:, :SLATE]


# speedup vs baseline: 2.1503x; 2.1503x over previous
import functools

import jax
import jax.numpy as jnp
from jax import lax
from jax.experimental import pallas as pl
from jax.experimental.pallas import tpu as pltpu
from jax.experimental.pallas import tpu_sc as plsc

SLATE = 50
B, H, V = 64, 8, 100000


VB = 8192
NB = 13
VP = NB * VB
NEG = -3.0e38


def _scores_body(x_ref, o_ref):
    x = x_ref[...]
    l = jnp.log(x + jnp.float32(1e-7))
    acc = l[:, 0]
    for h in range(1, H):
        acc = acc + l[:, h]
    col = pl.program_id(1) * VB + lax.broadcasted_iota(jnp.int32, (8, VB), 1)
    acc = jnp.where(col < V, acc, jnp.float32(NEG))
    o_ref[...] = jnp.transpose(acc.reshape(8, VB // 128, 128),
                               (1, 0, 2))[None]


def _scores(x, half, hb):
    return pl.pallas_call(
        _scores_body,
        grid=(hb // 8, NB),
        in_specs=[pl.BlockSpec(
            (8, H, VB), lambda g, j, half=half: (g + half * (hb // 8), 0, j))],
        out_specs=pl.BlockSpec(
            (1, VB // 128, 8, 128), lambda g, j: (g, j, 0, 0)),
        out_shape=jax.ShapeDtypeStruct((hb // 8, VP // 128, 8, 128),
                                       jnp.float32),
    )(x)


L = 16
NC, NS = 2, 16
ROWS_PER_W = B // (NC * NS)
CHUNKS = [(0, 1024, 8), (1024, 3072, 8), (4096, 20480, 8),
          (24576, 20480, 8), (45056, 20480, 8), (65536, 20480, 8),
          (86016, 20480, 8)]
DROWS = 160
CAP = 20480 + 64
IMIN = -2147483648
IMAX = 2147483647
KPAD = 64


def _splat(x, dtype=jnp.int32):
    return jnp.full((L,), x, dtype)


def _mono_key(v):
    bits = lax.bitcast_convert_type(v, jnp.int32)
    return jnp.where(bits < 0, IMIN - bits, bits)


def _count_pass(src, n, pivot, iota):
    nv = (n + L - 1) // L
    ps = jnp.broadcast_to(pivot, (L,))
    ns = jnp.broadcast_to(n, (L,))

    def body(i, carry):
        cg, ce = carry
        k = src[pl.ds(i * L, L)]
        valid = (iota + i * L) < ns
        g = (k > ps) & valid
        e = (k == ps) & valid
        return (cg + plsc.all_reduce_population_count(g),
                ce + plsc.all_reduce_population_count(e))

    cg, ce = lax.fori_loop(0, nv, body, (_splat(0), _splat(0)))
    return jnp.max(cg), jnp.max(ce)


def _filter_pass(src, dst, n, pivot, keep_gt, iota):
    nv = (n + L - 1) // L
    ps = jnp.broadcast_to(pivot, (L,))
    ns = jnp.broadcast_to(n, (L,))
    kgb = jnp.broadcast_to(keep_gt, (L,))

    def body(i, pos):
        k = src[pl.ds(i * L, L)]
        valid = (iota + i * L) < ns
        m = jnp.where(kgb, k > ps, k < ps) & valid
        pc = plsc.cumsum(m.astype(jnp.int32))
        tgt = pos + pc - 1
        plsc.store_scatter(dst, [tgt], k, mask=m)
        return pos + plsc.all_reduce_population_count(m)

    pos = lax.fori_loop(0, nv, body, _splat(0))
    return jnp.max(pos)


def _lane0(vec):
    iota = lax.iota(jnp.int32, L)
    return jnp.max(jnp.where(iota == 0, vec, _splat(IMIN)))


def _rebuild(cand_k, cand_i, surv, kept_k, kept_i, m, iota):
    need0 = jnp.int32(SLATE)

    def qs_iter(src, dst, n, need):
        pivot = _lane0(src[pl.ds(0, L)])
        cg, ce = _count_pass(src, n, pivot, iota)
        done = (cg < need) & (cg + ce >= need)
        keep_gt = cg >= need
        new_need = jnp.where(keep_gt, need, need - cg - ce)
        new_n = lax.cond(
            done, lambda _: n,
            lambda _: _filter_pass(src, dst, n, pivot, keep_gt, iota), 0)
        return new_n, new_need, pivot, done

    n1, need1, t1, done1 = qs_iter(cand_k, surv, m, need0)

    def w_cond(c):
        return jnp.logical_not(c[3])

    def w_body(c):
        n, need, _, _ = c
        return qs_iter(surv, surv, n, need)

    _, _, T, _ = lax.while_loop(w_cond, w_body, (n1, need1, t1, done1))

    r, _ = _count_pass(cand_k, m, T, iota)
    quota = jnp.broadcast_to(SLATE - r, (L,))
    Ts = jnp.broadcast_to(T, (L,))
    ms = jnp.broadcast_to(m, (L,))
    nv = (m + L - 1) // L

    def comp_body(i, carry):
        pos, eqc = carry
        k = cand_k[pl.ds(i * L, L)]
        ix = cand_i[pl.ds(i * L, L)]
        valid = (iota + i * L) < ms
        g = (k > Ts) & valid
        e = (k == Ts) & valid
        eseq = eqc + plsc.cumsum(e.astype(jnp.int32))
        keep = g | (e & (eseq <= quota))
        pc = plsc.cumsum(keep.astype(jnp.int32))
        tgt = pos + pc - 1
        plsc.store_scatter(kept_k, [tgt], k, mask=keep)
        plsc.store_scatter(kept_i, [tgt], ix, mask=keep)
        return (pos + plsc.all_reduce_population_count(keep),
                eqc + plsc.all_reduce_population_count(e))

    lax.fori_loop(0, nv, comp_body, (_splat(0), _splat(0)))
    plsc.store_scatter(kept_k, [iota + 48], _splat(IMIN), mask=iota >= 2)
    return T


def _topk_sc_body(rows_per_w, scores_hbm, out_hbm, data_a, data_b,
                  cand_k, cand_i, surv, kept_k, kept_i, outb, sem_a, sem_b):
    wid = lax.axis_index("s") * NC + lax.axis_index("c")
    iota = lax.iota(jnp.int32, L)
    bufs = (data_a, data_b)
    sems = (sem_a, sem_b)

    def row_body(rr, carry):
        row = wid * rows_per_w + rr
        for j in range(KPAD // L):
            kept_k[pl.ds(L * j, L)] = _splat(IMIN)
            kept_i[pl.ds(L * j, L)] = _splat(0)
        T = IMIN
        rg = row // 8
        rs = row % 8
        off0, size0, _ = CHUNKS[0]
        h = pltpu.async_copy(
            scores_hbm.at[rg, pl.ds(off0 // 128, size0 // 128),
                          pl.ds(rs, 1), :],
            bufs[0].at[pl.ds(0, size0 // 128), :, :], sems[0])
        for ci, (off, size, unroll) in enumerate(CHUNKS):
            data = bufs[ci % 2]
            if ci + 1 < len(CHUNKS):
                noff, nsize, _ = CHUNKS[ci + 1]
                hn = pltpu.async_copy(
                    scores_hbm.at[rg, pl.ds(noff // 128, nsize // 128),
                                  pl.ds(rs, 1), :],
                    bufs[(ci + 1) % 2].at[pl.ds(0, nsize // 128), :, :],
                    sems[(ci + 1) % 2])
            h.wait()
            for j in range(KPAD // L):
                cand_k[pl.ds(L * j, L)] = kept_k[pl.ds(L * j, L)]
                cand_i[pl.ds(L * j, L)] = kept_i[pl.ds(L * j, L)]
            Ts = jnp.broadcast_to(T, (L,))

            def app_body(i, pos, off=off, data=data, Ts=Ts, unroll=unroll):
                base = i * (L * unroll)
                for u in range(unroll):
                    f = i * (unroll // 8) + u // 8
                    v = data[f, 0, pl.ds((u % 8) * L, L)]
                    key = _mono_key(v)
                    anyh = plsc.all_reduce_population_count(key > Ts) > 0
                    tgt = pos + iota
                    plsc.store_scatter(cand_k, [tgt], key)
                    gidx = iota + (off + u * L) + base
                    plsc.store_scatter(cand_i, [tgt], gidx)
                    pos = pos + jnp.where(anyh, L, 0)
                return pos

            pos = lax.fori_loop(0, size // (L * unroll), app_body,
                                _splat(KPAD))
            m = jnp.max(pos)
            T = _rebuild(cand_k, cand_i, surv, kept_k, kept_i, m, iota)
            if ci + 1 < len(CHUNKS):
                h = hn

        ks = [kept_k[pl.ds(L * j, L)] for j in range(KPAD // L)]
        ix = [kept_i[pl.ds(L * j, L)] for j in range(KPAD // L)]
        for t in range(SLATE):
            m4 = jnp.maximum(jnp.maximum(ks[0], ks[1]),
                             jnp.maximum(ks[2], ks[3]))
            Mv = jnp.max(m4)
            Ms = jnp.broadcast_to(Mv, (L,))
            cands = [jnp.where(ks[j] == Ms, ix[j], _splat(IMAX))
                     for j in range(KPAD // L)]
            mn = jnp.min(jnp.minimum(jnp.minimum(cands[0], cands[1]),
                                     jnp.minimum(cands[2], cands[3])))
            mns = jnp.broadcast_to(mn, (L,))
            plsc.store_scatter(outb, [_splat(t)], mns, mask=iota == 0)
            ks = [jnp.where((ks[j] == Ms) & (ix[j] == mns), _splat(IMIN),
                            ks[j]) for j in range(KPAD // L)]
        plsc.store_scatter(outb, [iota + 48], _splat(0), mask=iota >= 2)
        pltpu.sync_copy(outb, out_hbm.at[pl.ds(row * KPAD, KPAD)])
        return carry

    lax.fori_loop(0, rows_per_w, row_body, jnp.int32(0))


def _topk_sc(scores, nrows):
    mesh = plsc.VectorSubcoreMesh(core_axis_name="c", subcore_axis_name="s")
    f = functools.partial(
        pl.kernel, mesh=mesh,
        compiler_params=pltpu.CompilerParams(needs_layout_passes=False),
        out_type=jax.ShapeDtypeStruct((nrows * KPAD,), jnp.int32),
        scratch_types=[
            pltpu.VMEM((DROWS, 1, 128), jnp.float32),
            pltpu.VMEM((DROWS, 1, 128), jnp.float32),
            pltpu.VMEM((CAP,), jnp.int32),
            pltpu.VMEM((CAP,), jnp.int32),
            pltpu.VMEM((CAP,), jnp.int32),
            pltpu.VMEM((KPAD,), jnp.int32),
            pltpu.VMEM((KPAD,), jnp.int32),
            pltpu.VMEM((KPAD,), jnp.int32),
            pltpu.SemaphoreType.DMA,
            pltpu.SemaphoreType.DMA,
        ],
    )(functools.partial(_topk_sc_body, nrows // (NC * NS)))
    return f(scores)


def kernel(batch_k_head_softmax):
    hb = B // 2
    s0 = _scores(batch_k_head_softmax, 0, hb)
    o0 = _topk_sc(s0, hb)
    s1 = _scores(batch_k_head_softmax, 1, hb)
    o1 = _topk_sc(s1, hb)
    out = jnp.concatenate([o0.reshape(hb, KPAD), o1.reshape(hb, KPAD)], 0)
    return out[:, :SLATE]


# speedup vs baseline: 2.1925x; 1.0196x over previous
import functools

import jax
import jax.numpy as jnp
from jax import lax
from jax.experimental import pallas as pl
from jax.experimental.pallas import tpu as pltpu
from jax.experimental.pallas import tpu_sc as plsc

SLATE = 50
B, H, V = 64, 8, 100000


VB = 8192
NB = 13
VP = NB * VB
NEG = -3.0e38


def _scores_body(x_ref, o_ref):
    x = x_ref[...]
    l = jnp.log(x + jnp.float32(1e-7))
    acc = l[:, 0]
    for h in range(1, H):
        acc = acc + l[:, h]
    col = pl.program_id(1) * VB + lax.broadcasted_iota(jnp.int32, (8, VB), 1)
    acc = jnp.where(col < V, acc, jnp.float32(NEG))
    o_ref[...] = jnp.transpose(acc.reshape(8, VB // 128, 128),
                               (1, 0, 2))[None]


def _scores(x, half, hb):
    return pl.pallas_call(
        _scores_body,
        grid=(hb // 8, NB),
        in_specs=[pl.BlockSpec(
            (8, H, VB), lambda g, j, half=half: (g + half * (hb // 8), 0, j))],
        out_specs=pl.BlockSpec(
            (1, VB // 128, 8, 128), lambda g, j: (g, j, 0, 0)),
        out_shape=jax.ShapeDtypeStruct((hb // 8, VP // 128, 8, 128),
                                       jnp.float32),
    )(x)


L = 16
NC, NS = 2, 16
ROWS_PER_W = B // (NC * NS)
CHUNKS = [(0, 1024, 8), (1024, 3072, 8), (4096, 8192, 8),
          (12288, 23552, 8), (35840, 23552, 8), (59392, 23552, 8),
          (82944, 23552, 8)]
DROWS = 184
CAP = 23552 + 64
IMIN = -2147483648
IMAX = 2147483647
KPAD = 64


def _splat(x, dtype=jnp.int32):
    return jnp.full((L,), x, dtype)


def _mono_key(v):
    bits = lax.bitcast_convert_type(v, jnp.int32)
    return jnp.where(bits < 0, IMIN - bits, bits)


def _count_pass(src, n, pivot, iota):
    nv = (n + L - 1) // L
    ps = jnp.broadcast_to(pivot, (L,))
    ns = jnp.broadcast_to(n, (L,))

    def body(i, carry):
        cg, ce = carry
        k = src[pl.ds(i * L, L)]
        valid = (iota + i * L) < ns
        g = (k > ps) & valid
        e = (k == ps) & valid
        return (cg + plsc.all_reduce_population_count(g),
                ce + plsc.all_reduce_population_count(e))

    cg, ce = lax.fori_loop(0, nv, body, (_splat(0), _splat(0)))
    return jnp.max(cg), jnp.max(ce)


def _filter_pass(src, dst, n, pivot, keep_gt, iota):
    nv = (n + L - 1) // L
    ps = jnp.broadcast_to(pivot, (L,))
    ns = jnp.broadcast_to(n, (L,))
    kgb = jnp.broadcast_to(keep_gt, (L,))

    def body(i, pos):
        k = src[pl.ds(i * L, L)]
        valid = (iota + i * L) < ns
        m = jnp.where(kgb, k > ps, k < ps) & valid
        pc = plsc.cumsum(m.astype(jnp.int32))
        tgt = pos + pc - 1
        plsc.store_scatter(dst, [tgt], k, mask=m)
        return pos + plsc.all_reduce_population_count(m)

    pos = lax.fori_loop(0, nv, body, _splat(0))
    return jnp.max(pos)


def _lane0(vec):
    iota = lax.iota(jnp.int32, L)
    return jnp.max(jnp.where(iota == 0, vec, _splat(IMIN)))


def _rebuild(cand_k, cand_i, surv, kept_k, kept_i, m, iota):
    need0 = jnp.int32(SLATE)

    def qs_iter(src, dst, n, need):
        pivot = _lane0(src[pl.ds(0, L)])
        cg, ce = _count_pass(src, n, pivot, iota)
        done = (cg < need) & (cg + ce >= need)
        keep_gt = cg >= need
        new_need = jnp.where(keep_gt, need, need - cg - ce)
        new_n = lax.cond(
            done, lambda _: n,
            lambda _: _filter_pass(src, dst, n, pivot, keep_gt, iota), 0)
        return new_n, new_need, pivot, done

    n1, need1, t1, done1 = qs_iter(cand_k, surv, m, need0)

    def w_cond(c):
        return jnp.logical_not(c[3])

    def w_body(c):
        n, need, _, _ = c
        return qs_iter(surv, surv, n, need)

    _, _, T, _ = lax.while_loop(w_cond, w_body, (n1, need1, t1, done1))

    r, _ = _count_pass(cand_k, m, T, iota)
    quota = jnp.broadcast_to(SLATE - r, (L,))
    Ts = jnp.broadcast_to(T, (L,))
    ms = jnp.broadcast_to(m, (L,))
    nv = (m + L - 1) // L

    def comp_body(i, carry):
        pos, eqc = carry
        k = cand_k[pl.ds(i * L, L)]
        ix = cand_i[pl.ds(i * L, L)]
        valid = (iota + i * L) < ms
        g = (k > Ts) & valid
        e = (k == Ts) & valid
        eseq = eqc + plsc.cumsum(e.astype(jnp.int32))
        keep = g | (e & (eseq <= quota))
        pc = plsc.cumsum(keep.astype(jnp.int32))
        tgt = pos + pc - 1
        plsc.store_scatter(kept_k, [tgt], k, mask=keep)
        plsc.store_scatter(kept_i, [tgt], ix, mask=keep)
        return (pos + plsc.all_reduce_population_count(keep),
                eqc + plsc.all_reduce_population_count(e))

    lax.fori_loop(0, nv, comp_body, (_splat(0), _splat(0)))
    plsc.store_scatter(kept_k, [iota + 48], _splat(IMIN), mask=iota >= 2)
    return T


def _topk_sc_body(rows_per_w, scores_hbm, out_hbm, data_a, data_b,
                  cand_k, cand_i, surv, kept_k, kept_i, outb, sem_a, sem_b):
    wid = lax.axis_index("s") * NC + lax.axis_index("c")
    iota = lax.iota(jnp.int32, L)
    bufs = (data_a, data_b)
    sems = (sem_a, sem_b)

    def row_body(rr, carry):
        row = wid * rows_per_w + rr
        for j in range(KPAD // L):
            kept_k[pl.ds(L * j, L)] = _splat(IMIN)
            kept_i[pl.ds(L * j, L)] = _splat(0)
        T = IMIN
        rg = row // 8
        rs = row % 8
        off0, size0, _ = CHUNKS[0]
        h = pltpu.async_copy(
            scores_hbm.at[rg, pl.ds(off0 // 128, size0 // 128),
                          pl.ds(rs, 1), :],
            bufs[0].at[pl.ds(0, size0 // 128), :, :], sems[0])
        for ci, (off, size, unroll) in enumerate(CHUNKS):
            data = bufs[ci % 2]
            if ci + 1 < len(CHUNKS):
                noff, nsize, _ = CHUNKS[ci + 1]
                hn = pltpu.async_copy(
                    scores_hbm.at[rg, pl.ds(noff // 128, nsize // 128),
                                  pl.ds(rs, 1), :],
                    bufs[(ci + 1) % 2].at[pl.ds(0, nsize // 128), :, :],
                    sems[(ci + 1) % 2])
            h.wait()
            for j in range(KPAD // L):
                cand_k[pl.ds(L * j, L)] = kept_k[pl.ds(L * j, L)]
                cand_i[pl.ds(L * j, L)] = kept_i[pl.ds(L * j, L)]
            Ts = jnp.broadcast_to(T, (L,))

            def app_body(i, pos, off=off, data=data, Ts=Ts, unroll=unroll):
                base = i * (L * unroll)
                for u in range(unroll):
                    f = i * (unroll // 8) + u // 8
                    v = data[f, 0, pl.ds((u % 8) * L, L)]
                    key = _mono_key(v)
                    anyh = plsc.all_reduce_population_count(key > Ts) > 0
                    tgt = pos + iota
                    plsc.store_scatter(cand_k, [tgt], key)
                    gidx = iota + (off + u * L) + base
                    plsc.store_scatter(cand_i, [tgt], gidx)
                    pos = pos + jnp.where(anyh, L, 0)
                return pos

            pos = lax.fori_loop(0, size // (L * unroll), app_body,
                                _splat(KPAD))
            m = jnp.max(pos)
            T = _rebuild(cand_k, cand_i, surv, kept_k, kept_i, m, iota)
            if ci + 1 < len(CHUNKS):
                h = hn

        ks = [kept_k[pl.ds(L * j, L)] for j in range(KPAD // L)]
        ix = [kept_i[pl.ds(L * j, L)] for j in range(KPAD // L)]
        for t in range(SLATE):
            m4 = jnp.maximum(jnp.maximum(ks[0], ks[1]),
                             jnp.maximum(ks[2], ks[3]))
            Mv = jnp.max(m4)
            Ms = jnp.broadcast_to(Mv, (L,))
            cands = [jnp.where(ks[j] == Ms, ix[j], _splat(IMAX))
                     for j in range(KPAD // L)]
            mn = jnp.min(jnp.minimum(jnp.minimum(cands[0], cands[1]),
                                     jnp.minimum(cands[2], cands[3])))
            mns = jnp.broadcast_to(mn, (L,))
            plsc.store_scatter(outb, [_splat(t)], mns, mask=iota == 0)
            ks = [jnp.where((ks[j] == Ms) & (ix[j] == mns), _splat(IMIN),
                            ks[j]) for j in range(KPAD // L)]
        plsc.store_scatter(outb, [iota + 48], _splat(0), mask=iota >= 2)
        pltpu.sync_copy(outb, out_hbm.at[pl.ds(row * KPAD, KPAD)])
        return carry

    lax.fori_loop(0, rows_per_w, row_body, jnp.int32(0))


def _topk_sc(scores, nrows):
    mesh = plsc.VectorSubcoreMesh(core_axis_name="c", subcore_axis_name="s")
    f = functools.partial(
        pl.kernel, mesh=mesh,
        compiler_params=pltpu.CompilerParams(needs_layout_passes=False),
        out_type=jax.ShapeDtypeStruct((nrows * KPAD,), jnp.int32),
        scratch_types=[
            pltpu.VMEM((DROWS, 1, 128), jnp.float32),
            pltpu.VMEM((DROWS, 1, 128), jnp.float32),
            pltpu.VMEM((CAP,), jnp.int32),
            pltpu.VMEM((CAP,), jnp.int32),
            pltpu.VMEM((CAP,), jnp.int32),
            pltpu.VMEM((KPAD,), jnp.int32),
            pltpu.VMEM((KPAD,), jnp.int32),
            pltpu.VMEM((KPAD,), jnp.int32),
            pltpu.SemaphoreType.DMA,
            pltpu.SemaphoreType.DMA,
        ],
    )(functools.partial(_topk_sc_body, nrows // (NC * NS)))
    return f(scores)


def kernel(batch_k_head_softmax):
    hb = B // 2
    s0 = _scores(batch_k_head_softmax, 0, hb)
    o0 = _topk_sc(s0, hb)
    s1 = _scores(batch_k_head_softmax, 1, hb)
    o1 = _topk_sc(s1, hb)
    out = jnp.concatenate([o0.reshape(hb, KPAD), o1.reshape(hb, KPAD)], 0)
    return out[:, :SLATE]
